# Initial kernel scaffold; baseline (speedup 1.0000x reference)
import jax
import jax.numpy as jnp
from jax.experimental import pallas as pl


def kernel(x, edge_index, W1, b1, g1, be1, W2, b2, eps, g2, be2, Wf, bf):
    raise NotImplementedError("write your pallas kernel here")


# speedup vs baseline: 7.4753x; 7.4753x over previous
import functools

import jax
import jax.numpy as jnp
from jax import lax
from jax.experimental import pallas as pl
from jax.experimental.pallas import tpu as pltpu
from jax.experimental.pallas import tpu_sc as plsc

N = 10000
E = 320000
D = 128

NC = 2
NS = 16
NW = NC * NS

C = 125
CPW = E // (NW * C)
RPS = (N // NS) // 8 * 8
REM = N - NS * RPS
ZR = 16

_mesh = plsc.VectorSubcoreMesh(core_axis_name="c", subcore_axis_name="s")


@functools.partial(
    pl.kernel,
    out_type=jax.ShapeDtypeStruct((NC, N, D), jnp.float32),
    mesh=_mesh,
    scratch_types=[
        pltpu.VMEM((CPW, C), jnp.int32),
        pltpu.VMEM((CPW, C), jnp.int32),
        pltpu.VMEM((C, D), jnp.float32),
        pltpu.VMEM((ZR, D), jnp.float32),
        pltpu.VMEM_SHARED((N, D), jnp.float32),
        pltpu.SemaphoreType.DMA,
    ],
)
def _sc_agg(h_hbm, src_hbm, dst_hbm, out_hbm, src_v, dst_v, rows_v, z_v,
            acc_sh, sem):
    cid = lax.axis_index("c")
    sid = lax.axis_index("s")
    wid = cid * NS + sid

    zero = jnp.zeros((16,), jnp.float32)
    for i in range(ZR):
        for j in range(D // 16):
            z_v[i, pl.ds(j * 16, 16)] = zero
    for k in range(RPS // ZR):
        pltpu.sync_copy(z_v, acc_sh.at[pl.ds(sid * RPS + k * ZR, ZR)])

    @pl.when(sid == 0)
    def _():
        pltpu.sync_copy(z_v.at[pl.ds(0, REM)], acc_sh.at[pl.ds(NS * RPS, REM)])

    plsc.subcore_barrier()

    pltpu.sync_copy(src_hbm.at[pl.ds(wid * CPW, CPW)], src_v)
    pltpu.sync_copy(dst_hbm.at[pl.ds(wid * CPW, CPW)], dst_v)

    def ebody(j, carry):
        pltpu.async_copy(h_hbm.at[src_v.at[j]], rows_v, sem).wait()
        pltpu.sync_copy(rows_v, acc_sh.at[dst_v.at[j]], add=True)
        return carry

    lax.fori_loop(0, CPW, ebody, 0)

    plsc.subcore_barrier()
    pltpu.sync_copy(acc_sh.at[pl.ds(sid * RPS, RPS)],
                    out_hbm.at[cid, pl.ds(sid * RPS, RPS)])

    @pl.when(sid == 0)
    def _():
        pltpu.sync_copy(acc_sh.at[pl.ds(NS * RPS, REM)],
                        out_hbm.at[cid, pl.ds(NS * RPS, REM)])


def _bn(z, g, b):
    m = jnp.mean(z, axis=0, keepdims=True)
    v = jnp.mean((z - m) * (z - m), axis=0, keepdims=True)
    return (z - m) * lax.rsqrt(v + 1e-5) * g + b


def _leaky(z):
    return jnp.where(z >= 0, z, 0.01 * z)


def _mlp_mid_body(h_ref, p_ref, w1_ref, b1_ref, g1_ref, be1_ref, w2_ref,
                  b2_ref, g2_ref, be2_ref, eps_ref, o_ref):
    z = h_ref[...] * eps_ref[...] + p_ref[0] + p_ref[1]
    z = jnp.dot(z, w1_ref[...], preferred_element_type=jnp.float32) + b1_ref[...]
    z = _leaky(_bn(z, g1_ref[...], be1_ref[...]))
    z = jnp.dot(z, w2_ref[...], preferred_element_type=jnp.float32) + b2_ref[...]
    o_ref[...] = _leaky(_bn(z, g2_ref[...], be2_ref[...]))


def _mlp_last_body(h_ref, p_ref, w1_ref, b1_ref, g1_ref, be1_ref, w2_ref,
                   b2_ref, wf_ref, bf_ref, eps_ref, o_ref):
    z = h_ref[...] * eps_ref[...] + p_ref[0] + p_ref[1]
    z = jnp.dot(z, w1_ref[...], preferred_element_type=jnp.float32) + b1_ref[...]
    z = _leaky(_bn(z, g1_ref[...], be1_ref[...]))
    z = jnp.dot(z, w2_ref[...], preferred_element_type=jnp.float32) + b2_ref[...]
    o_ref[...] = jnp.sum(z * wf_ref[...], axis=1, keepdims=True) + bf_ref[...]


_mlp_mid = pl.pallas_call(
    _mlp_mid_body,
    out_shape=jax.ShapeDtypeStruct((N, D), jnp.float32),
)

_mlp_last = pl.pallas_call(
    _mlp_last_body,
    out_shape=jax.ShapeDtypeStruct((N, 1), jnp.float32),
)


def kernel(x, edge_index, W1, b1, g1, be1, W2, b2, eps, g2, be2, Wf, bf):
    src = edge_index[0].astype(jnp.int32).reshape(NW * CPW, C)
    dst = edge_index[1].astype(jnp.int32).reshape(NW * CPW, C)
    L = W1.shape[0]
    h = x
    for l in range(L):
        parts = _sc_agg(h, src, dst)
        epsb = jnp.full((1, D), 1.0 + eps[l], jnp.float32)
        if l != L - 1:
            h = _mlp_mid(h, parts, W1[l], b1[l].reshape(1, D),
                         g1[l].reshape(1, D), be1[l].reshape(1, D), W2[l],
                         b2[l].reshape(1, D), g2[l].reshape(1, D),
                         be2[l].reshape(1, D), epsb)
        else:
            out = _mlp_last(h, parts, W1[l], b1[l].reshape(1, D),
                            g1[l].reshape(1, D), be1[l].reshape(1, D), W2[l],
                            b2[l].reshape(1, D), Wf.reshape(1, D),
                            bf.reshape(1, 1), epsb)
    return out.reshape(-1)


# speedup vs baseline: 8.7526x; 1.1709x over previous
import functools

import jax
import jax.numpy as jnp
from jax import lax
from jax.experimental import pallas as pl
from jax.experimental.pallas import tpu as pltpu
from jax.experimental.pallas import tpu_sc as plsc

N = 10000
E = 320000
D = 128

NC = 2
NS = 16
NW = NC * NS

C = 125
CPW = E // (NW * C)
PH = 2
HCH = CPW // PH
RPS = (N // NS) // 8 * 8
REM = N - NS * RPS
ZR = 16
NB = 2

_mesh = plsc.VectorSubcoreMesh(core_axis_name="c", subcore_axis_name="s")


@functools.partial(
    pl.kernel,
    out_type=jax.ShapeDtypeStruct((NC, N, D), jnp.float32),
    mesh=_mesh,
    scratch_types=[
        pltpu.VMEM((HCH, C), jnp.int32),
        pltpu.VMEM((HCH, C), jnp.int32),
        pltpu.VMEM((NB, C, D), jnp.float32),
        pltpu.VMEM((ZR, D), jnp.float32),
        pltpu.VMEM_SHARED((N, D), jnp.float32),
        [pltpu.SemaphoreType.DMA] * NB,
        [pltpu.SemaphoreType.DMA] * NB,
    ],
)
def _sc_agg(h_hbm, src_hbm, dst_hbm, out_hbm, src_v, dst_v, rows_v, z_v,
            acc_sh, gsems, ssems):
    cid = lax.axis_index("c")
    sid = lax.axis_index("s")
    wid = cid * NS + sid

    zero = jnp.zeros((16,), jnp.float32)
    for i in range(ZR):
        for j in range(D // 16):
            z_v[i, pl.ds(j * 16, 16)] = zero
    for k in range(RPS // ZR):
        pltpu.sync_copy(z_v, acc_sh.at[pl.ds(sid * RPS + k * ZR, ZR)])

    @pl.when(sid == 0)
    def _():
        pltpu.sync_copy(z_v.at[pl.ds(0, REM)], acc_sh.at[pl.ds(NS * RPS, REM)])

    plsc.subcore_barrier()

    for ph in range(PH):
        if ph > 0:
            for b in range(NB):
                pltpu.make_async_copy(
                    h_hbm.at[src_v.at[0]], rows_v.at[b], ssems[b]).wait()
        pltpu.sync_copy(src_hbm.at[pl.ds(wid * CPW + ph * HCH, HCH)], src_v)
        pltpu.sync_copy(dst_hbm.at[pl.ds(wid * CPW + ph * HCH, HCH)], dst_v)

        def gbody(g, carry):
            handles = []
            for b in range(NB):
                @pl.when(g > 0)
                def _(b=b):
                    pltpu.make_async_copy(
                        h_hbm.at[src_v.at[0]], rows_v.at[b], ssems[b]).wait()
                handles.append(pltpu.async_copy(
                    h_hbm.at[src_v.at[g * NB + b]], rows_v.at[b], gsems[b]))
            for b in range(NB):
                handles[b].wait()
                pltpu.async_copy(rows_v.at[b], acc_sh.at[dst_v.at[g * NB + b]],
                                 ssems[b], add=True)
            return carry

        lax.fori_loop(0, HCH // NB, gbody, 0)

    for b in range(NB):
        pltpu.make_async_copy(h_hbm.at[src_v.at[0]], rows_v.at[b],
                              ssems[b]).wait()

    plsc.subcore_barrier()
    pltpu.sync_copy(acc_sh.at[pl.ds(sid * RPS, RPS)],
                    out_hbm.at[cid, pl.ds(sid * RPS, RPS)])

    @pl.when(sid == 0)
    def _():
        pltpu.sync_copy(acc_sh.at[pl.ds(NS * RPS, REM)],
                        out_hbm.at[cid, pl.ds(NS * RPS, REM)])


def _bn(z, g, b):
    m = jnp.mean(z, axis=0, keepdims=True)
    v = jnp.mean((z - m) * (z - m), axis=0, keepdims=True)
    return (z - m) * lax.rsqrt(v + 1e-5) * g + b


def _leaky(z):
    return jnp.where(z >= 0, z, 0.01 * z)


def _mlp_mid_body(h_ref, p_ref, w1_ref, b1_ref, g1_ref, be1_ref, w2_ref,
                  b2_ref, g2_ref, be2_ref, eps_ref, o_ref):
    z = h_ref[...] * eps_ref[...] + p_ref[0] + p_ref[1]
    z = jnp.dot(z, w1_ref[...], preferred_element_type=jnp.float32) + b1_ref[...]
    z = _leaky(_bn(z, g1_ref[...], be1_ref[...]))
    z = jnp.dot(z, w2_ref[...], preferred_element_type=jnp.float32) + b2_ref[...]
    o_ref[...] = _leaky(_bn(z, g2_ref[...], be2_ref[...]))


def _mlp_last_body(h_ref, p_ref, w1_ref, b1_ref, g1_ref, be1_ref, w2_ref,
                   b2_ref, wf_ref, bf_ref, eps_ref, o_ref):
    z = h_ref[...] * eps_ref[...] + p_ref[0] + p_ref[1]
    z = jnp.dot(z, w1_ref[...], preferred_element_type=jnp.float32) + b1_ref[...]
    z = _leaky(_bn(z, g1_ref[...], be1_ref[...]))
    z = jnp.dot(z, w2_ref[...], preferred_element_type=jnp.float32) + b2_ref[...]
    o_ref[...] = jnp.sum(z * wf_ref[...], axis=1, keepdims=True) + bf_ref[...]


_mlp_mid = pl.pallas_call(
    _mlp_mid_body,
    out_shape=jax.ShapeDtypeStruct((N, D), jnp.float32),
)

_mlp_last = pl.pallas_call(
    _mlp_last_body,
    out_shape=jax.ShapeDtypeStruct((N, 1), jnp.float32),
)


def kernel(x, edge_index, W1, b1, g1, be1, W2, b2, eps, g2, be2, Wf, bf):
    src = edge_index[0].astype(jnp.int32).reshape(NW * CPW, C)
    dst = edge_index[1].astype(jnp.int32).reshape(NW * CPW, C)
    L = W1.shape[0]
    h = x
    for l in range(L):
        parts = _sc_agg(h, src, dst)
        epsb = jnp.full((1, D), 1.0 + eps[l], jnp.float32)
        if l != L - 1:
            h = _mlp_mid(h, parts, W1[l], b1[l].reshape(1, D),
                         g1[l].reshape(1, D), be1[l].reshape(1, D), W2[l],
                         b2[l].reshape(1, D), g2[l].reshape(1, D),
                         be2[l].reshape(1, D), epsb)
        else:
            out = _mlp_last(h, parts, W1[l], b1[l].reshape(1, D),
                            g1[l].reshape(1, D), be1[l].reshape(1, D), W2[l],
                            b2[l].reshape(1, D), Wf.reshape(1, D),
                            bf.reshape(1, 1), epsb)
    return out.reshape(-1)
